# Initial kernel scaffold; baseline (speedup 1.0000x reference)
import jax
import jax.numpy as jnp
from jax.experimental import pallas as pl


def kernel(xl0, xp0, edge_index_l, edge_index_p, eps_l, eps_p, W0_l, Wm_l, Ws_l, W0_p, Wm_p, Ws_p):
    raise NotImplementedError("write your pallas kernel here")


# speedup vs baseline: 9.8368x; 9.8368x over previous
import functools

import jax
import jax.numpy as jnp
from jax import lax
from jax.experimental import pallas as pl
from jax.experimental.pallas import tpu as pltpu
from jax.experimental.pallas import tpu_sc as plsc

N = 10000
NPAD = 10240
E = 320000
EPAD = 327680
D = 128
H1 = 256
H2 = 64
NC, NS = 2, 16
BLK = 128

_MESH = dict(core_axis_name="c", subcore_axis_name="s", num_cores=NC,
             num_subcores=NS)
_SC_PARAMS = pltpu.CompilerParams(needs_layout_passes=False)


def _deg_kernel(dst_a_hbm, dst_b_hbm, out_a_hbm, out_b_hbm, hist, idx):
    cid = lax.axis_index("c")
    sid = lax.axis_index("s")
    wid = cid * NS + sid
    ones = jnp.full((16,), 1.0, jnp.float32)
    zeros = jnp.zeros((16,), jnp.float32)

    def one_graph(dst_hbm, out_hbm):
        def zero(i, _):
            hist[pl.ds(i * 16, 16)] = zeros
            return 0
        lax.fori_loop(0, NPAD // 16, zero, 0)
        base = wid * (EPAD // (NC * NS))

        def block(b, _):
            pltpu.sync_copy(dst_hbm.at[pl.ds(base + b * 512, 512)], idx)
            for j in range(32):
                iv = idx[pl.ds(j * 16, 16)]
                plsc.addupdate_scatter(hist, [iv], ones)
            return 0
        lax.fori_loop(0, EPAD // (NC * NS) // 512, block, 0)
        pltpu.sync_copy(hist, out_hbm.at[wid])

    one_graph(dst_a_hbm, out_a_hbm)
    one_graph(dst_b_hbm, out_b_hbm)


def _deg(dst_a, dst_b):
    return pl.kernel(
        _deg_kernel,
        out_type=(jax.ShapeDtypeStruct((NC * NS, NPAD), jnp.float32),) * 2,
        mesh=plsc.VectorSubcoreMesh(**_MESH),
        scratch_types=[pltpu.VMEM((NPAD,), jnp.float32),
                       pltpu.VMEM((512,), jnp.int32)],
        compiler_params=_SC_PARAMS,
    )(dst_a, dst_b)


def _edge_loop(src_hbm, dst_hbm, h_hbm, acc, idxs, idxd, rows, gsem,
               base, nblk):
    def block(b, _):
        off = base + b * BLK
        pltpu.sync_copy(src_hbm.at[pl.ds(off, BLK)], idxs)
        pltpu.sync_copy(dst_hbm.at[pl.ds(off, BLK)], idxd)
        pltpu.async_copy(h_hbm.at[idxs], rows, gsem).wait()
        pltpu.sync_copy(rows, acc.at[idxd], add=True)
        return 0
    lax.fori_loop(0, nblk, block, 0)


def _prop_col_kernel(h0_hbm, h1_hbm, src_hbm, dst_hbm, z_hbm,
                     o0_hbm, o1_hbm, acc, idxs, idxd, rows, gsem):
    cid = lax.axis_index("c")
    sid = lax.axis_index("s")
    rows_per = NPAD // NS
    pltpu.sync_copy(z_hbm, acc.at[pl.ds(sid * rows_per, rows_per)])
    plsc.subcore_barrier()
    base = sid * (EPAD // NS)
    nblk = EPAD // NS // BLK

    @pl.when(cid == 0)
    def _():
        _edge_loop(src_hbm, dst_hbm, h0_hbm, acc, idxs, idxd, rows, gsem,
                   base, nblk)

    @pl.when(cid == 1)
    def _():
        _edge_loop(src_hbm, dst_hbm, h1_hbm, acc, idxs, idxd, rows, gsem,
                   base, nblk)

    plsc.subcore_barrier()
    sl = pl.ds(sid * rows_per, rows_per)

    @pl.when(cid == 0)
    def _():
        pltpu.sync_copy(acc.at[sl], o0_hbm.at[sl])

    @pl.when(cid == 1)
    def _():
        pltpu.sync_copy(acc.at[sl], o1_hbm.at[sl])


def _prop_edge_kernel(h_hbm, src_hbm, dst_hbm, z_hbm,
                      o0_hbm, o1_hbm, acc, idxs, idxd, rows, gsem):
    cid = lax.axis_index("c")
    sid = lax.axis_index("s")
    rows_per = NPAD // NS
    pltpu.sync_copy(z_hbm, acc.at[pl.ds(sid * rows_per, rows_per)])
    plsc.subcore_barrier()
    wid = cid * NS + sid
    base = wid * (EPAD // (NC * NS))
    nblk = EPAD // (NC * NS) // BLK
    _edge_loop(src_hbm, dst_hbm, h_hbm, acc, idxs, idxd, rows, gsem,
               base, nblk)
    plsc.subcore_barrier()
    sl = pl.ds(sid * rows_per, rows_per)

    @pl.when(cid == 0)
    def _():
        pltpu.sync_copy(acc.at[sl], o0_hbm.at[sl])

    @pl.when(cid == 1)
    def _():
        pltpu.sync_copy(acc.at[sl], o1_hbm.at[sl])


def _prop_scratch():
    return [pltpu.VMEM_SHARED((NPAD, 128), jnp.float32),
            pltpu.VMEM((BLK,), jnp.int32),
            pltpu.VMEM((BLK,), jnp.int32),
            pltpu.VMEM((BLK, 128), jnp.float32),
            pltpu.SemaphoreType.DMA]


def _prop_col(h0, h1, src, dst, zrows):
    return pl.kernel(
        _prop_col_kernel,
        out_type=(jax.ShapeDtypeStruct((NPAD, 128), jnp.float32),) * 2,
        mesh=plsc.VectorSubcoreMesh(**_MESH),
        scratch_types=_prop_scratch(),
        compiler_params=_SC_PARAMS,
    )(h0, h1, src, dst, zrows)


def _prop_edge(h, src, dst, zrows):
    return pl.kernel(
        _prop_edge_kernel,
        out_type=(jax.ShapeDtypeStruct((NPAD, 128), jnp.float32),) * 2,
        mesh=plsc.VectorSubcoreMesh(**_MESH),
        scratch_types=_prop_scratch(),
        compiler_params=_SC_PARAMS,
    )(h, src, dst, zrows)


RB = 256


def _rdeg(degp):
    return lax.rsqrt(jnp.maximum(jnp.sum(degp, axis=0), 1.0))


def _tc1_body(x_ref, w_ref, degp_ref, o0_ref, o1_ref):
    rdeg = _rdeg(degp_ref[...])
    h = jnp.dot(x_ref[...], w_ref[...], preferred_element_type=jnp.float32)
    hp = h * rdeg[:, None]
    o0_ref[...] = hp[:, :128]
    o1_ref[...] = hp[:, 128:]


def _tc1(x, w, degp):
    return pl.pallas_call(
        _tc1_body,
        grid=(NPAD // RB,),
        in_specs=[pl.BlockSpec((RB, D), lambda i: (i, 0)),
                  pl.BlockSpec((D, H1), lambda i: (0, 0)),
                  pl.BlockSpec((NC * NS, RB), lambda i: (0, i))],
        out_specs=[pl.BlockSpec((RB, 128), lambda i: (i, 0)),
                   pl.BlockSpec((RB, 128), lambda i: (i, 0))],
        out_shape=[jax.ShapeDtypeStruct((NPAD, 128), jnp.float32)] * 2,
    )(x, w, degp)


def _tc2_body(p0_ref, p1_ref, w_ref, degp_ref, o_ref):
    rdeg = _rdeg(degp_ref[...])
    hidden = jnp.concatenate([p0_ref[...], p1_ref[...]], axis=1)
    hidden = jnp.maximum(hidden * rdeg[:, None], 0.0)
    o_ref[...] = jnp.dot(hidden, w_ref[...],
                         preferred_element_type=jnp.float32) * rdeg[:, None]


def _tc2(p0, p1, w, degp):
    return pl.pallas_call(
        _tc2_body,
        grid=(NPAD // RB,),
        in_specs=[pl.BlockSpec((RB, 128), lambda i: (i, 0)),
                  pl.BlockSpec((RB, 128), lambda i: (i, 0)),
                  pl.BlockSpec((H1, 2 * H2), lambda i: (0, 0)),
                  pl.BlockSpec((NC * NS, RB), lambda i: (0, i))],
        out_specs=pl.BlockSpec((RB, 2 * H2), lambda i: (i, 0)),
        out_shape=jax.ShapeDtypeStruct((NPAD, 2 * H2), jnp.float32),
    )(p0, p1, w, degp)


def _tc3_body(q0_ref, q1_ref, degp_ref, eps_ref, mean_ref, std_ref, z_ref):
    rdeg = _rdeg(degp_ref[...])
    p = (q0_ref[...] + q1_ref[...]) * rdeg[:, None]
    mean = p[:, :H2]
    std = jnp.exp(p[:, H2:])
    mean_ref[...] = mean
    std_ref[...] = std
    z_ref[...] = mean + eps_ref[...] * std


def _tc3(q0, q1, degp, eps):
    return pl.pallas_call(
        _tc3_body,
        grid=(NPAD // RB,),
        in_specs=[pl.BlockSpec((RB, 128), lambda i: (i, 0)),
                  pl.BlockSpec((RB, 128), lambda i: (i, 0)),
                  pl.BlockSpec((NC * NS, RB), lambda i: (0, i)),
                  pl.BlockSpec((RB, H2), lambda i: (i, 0))],
        out_specs=[pl.BlockSpec((RB, H2), lambda i: (i, 0))] * 3,
        out_shape=[jax.ShapeDtypeStruct((NPAD, H2), jnp.float32)] * 3,
    )(q0, q1, degp, eps)


def _pad_edges(edge_index):
    pad = N + (jnp.arange(EPAD - E, dtype=jnp.int32) % (NPAD - N))
    src = jnp.concatenate([edge_index[0].astype(jnp.int32), pad])
    dst = jnp.concatenate([edge_index[1].astype(jnp.int32), pad])
    return src, dst


def _encoder(x, src, dst, degp, eps, W0, Wcat, zrows):
    xpad = jnp.pad(x, ((0, NPAD - N), (0, 0)))
    h1c0, h1c1 = _tc1(xpad, W0, degp)
    p1c0, p1c1 = _prop_col(h1c0, h1c1, src, dst, zrows)
    h2 = _tc2(p1c0, p1c1, Wcat, degp)
    q0, q1 = _prop_edge(h2, src, dst, zrows)
    epspad = jnp.pad(eps, ((0, NPAD - N), (0, 0)))
    mean, std, z = _tc3(q0, q1, degp, epspad)
    return mean[:N], std[:N], z[:N]


def kernel(xl0, xp0, edge_index_l, edge_index_p, eps_l, eps_p,
           W0_l, Wm_l, Ws_l, W0_p, Wm_p, Ws_p):
    src_l, dst_l = _pad_edges(edge_index_l)
    src_p, dst_p = _pad_edges(edge_index_p)
    zrows = jnp.zeros((NPAD // NS, 128), jnp.float32)
    degp_l, degp_p = _deg(dst_l, dst_p)
    Wcat_l = jnp.concatenate([Wm_l, Ws_l], axis=1)
    Wcat_p = jnp.concatenate([Wm_p, Ws_p], axis=1)
    hl, stdl, xl = _encoder(xl0, src_l, dst_l, degp_l, eps_l, W0_l, Wcat_l,
                            zrows)
    hp, stdp, xp = _encoder(xp0, src_p, dst_p, degp_p, eps_p, W0_p, Wcat_p,
                            zrows)
    return (hl, stdl, xl, hp, stdp, xp)


# speedup vs baseline: 14.0780x; 1.4312x over previous
import functools

import jax
import jax.numpy as jnp
from jax import lax
from jax.experimental import pallas as pl
from jax.experimental.pallas import tpu as pltpu
from jax.experimental.pallas import tpu_sc as plsc

N = 10000
NPAD = 10240
E = 320000
EPAD = 327680
D = 128
H1 = 256
H2 = 64
NC, NS = 2, 16
BLK = 128

_MESH = dict(core_axis_name="c", subcore_axis_name="s", num_cores=NC,
             num_subcores=NS)
_SC_PARAMS = pltpu.CompilerParams(needs_layout_passes=False)


def _deg_kernel(dst_a_hbm, dst_b_hbm, out_a_hbm, out_b_hbm, hist, idx):
    cid = lax.axis_index("c")
    sid = lax.axis_index("s")
    wid = cid * NS + sid
    ones = jnp.full((16,), 1.0, jnp.float32)
    zeros = jnp.zeros((16,), jnp.float32)

    def one_graph(dst_hbm, out_hbm):
        def zero(i, _):
            hist[pl.ds(i * 16, 16)] = zeros
            return 0
        lax.fori_loop(0, NPAD // 16, zero, 0)
        base = wid * (EPAD // (NC * NS))

        def block(b, _):
            pltpu.sync_copy(dst_hbm.at[pl.ds(base + b * 512, 512)], idx)
            for j in range(32):
                iv = idx[pl.ds(j * 16, 16)]
                plsc.addupdate_scatter(hist, [iv], ones)
            return 0
        lax.fori_loop(0, EPAD // (NC * NS) // 512, block, 0)
        pltpu.sync_copy(hist, out_hbm.at[wid])

    one_graph(dst_a_hbm, out_a_hbm)
    one_graph(dst_b_hbm, out_b_hbm)


def _deg(dst_a, dst_b):
    return pl.kernel(
        _deg_kernel,
        out_type=(jax.ShapeDtypeStruct((NC * NS, NPAD), jnp.float32),) * 2,
        mesh=plsc.VectorSubcoreMesh(**_MESH),
        scratch_types=[pltpu.VMEM((NPAD,), jnp.float32),
                       pltpu.VMEM((512,), jnp.int32)],
        compiler_params=_SC_PARAMS,
    )(dst_a, dst_b)


def _edge_loop(src_hbm, dst_hbm, h_hbm, acc, bufs, base, nblk):
    (idxs, idxd, rows, isem, gsem, ssem) = bufs

    def prefetch(b, p):
        off = base + b * BLK
        pltpu.async_copy(src_hbm.at[pl.ds(off, BLK)], idxs[p], isem[p])
        pltpu.async_copy(dst_hbm.at[pl.ds(off, BLK)], idxd[p], isem[p])

    def wait_prefetch(p):
        pltpu.make_async_copy(src_hbm.at[pl.ds(base, BLK)], idxs[p],
                              isem[p]).wait()
        pltpu.make_async_copy(dst_hbm.at[pl.ds(base, BLK)], idxd[p],
                              isem[p]).wait()

    def wait_scatter(p):
        pltpu.make_async_copy(rows[p], acc.at[idxd[p]], ssem[p]).wait()

    def block_step(b, p):
        q = 1 - p
        wait_prefetch(p)
        pltpu.async_copy(h_hbm.at[idxs[p]], rows[p], gsem[p]).wait()

        @pl.when(b >= 1)
        def _():
            wait_scatter(q)

        @pl.when(b + 1 < nblk)
        def _():
            prefetch(b + 1, q)

        pltpu.async_copy(rows[p], acc.at[idxd[p]], ssem[p], add=True)

    prefetch(0, 0)

    def pair(m, _):
        block_step(2 * m, 0)
        block_step(2 * m + 1, 1)
        return 0
    lax.fori_loop(0, nblk // 2, pair, 0)
    wait_scatter((nblk - 1) % 2)


def _prop_col_kernel(h0_hbm, h1_hbm, src_hbm, dst_hbm, z_hbm,
                     o0_hbm, o1_hbm, acc, idxs0, idxs1, idxd0, idxd1,
                     rows0, rows1, isem0, isem1, gsem0, gsem1, ssem0, ssem1):
    cid = lax.axis_index("c")
    sid = lax.axis_index("s")
    bufs = ((idxs0, idxs1), (idxd0, idxd1), (rows0, rows1),
            (isem0, isem1), (gsem0, gsem1), (ssem0, ssem1))
    rows_per = NPAD // NS
    pltpu.sync_copy(z_hbm, acc.at[pl.ds(sid * rows_per, rows_per)])
    plsc.subcore_barrier()
    base = sid * (EPAD // NS)
    nblk = EPAD // NS // BLK

    @pl.when(cid == 0)
    def _():
        _edge_loop(src_hbm, dst_hbm, h0_hbm, acc, bufs, base, nblk)

    @pl.when(cid == 1)
    def _():
        _edge_loop(src_hbm, dst_hbm, h1_hbm, acc, bufs, base, nblk)

    plsc.subcore_barrier()
    sl = pl.ds(sid * rows_per, rows_per)

    @pl.when(cid == 0)
    def _():
        pltpu.sync_copy(acc.at[sl], o0_hbm.at[sl])

    @pl.when(cid == 1)
    def _():
        pltpu.sync_copy(acc.at[sl], o1_hbm.at[sl])


def _prop_edge_kernel(h_hbm, src_hbm, dst_hbm, z_hbm,
                      o0_hbm, o1_hbm, acc, idxs0, idxs1, idxd0, idxd1,
                      rows0, rows1, isem0, isem1, gsem0, gsem1, ssem0, ssem1):
    cid = lax.axis_index("c")
    sid = lax.axis_index("s")
    bufs = ((idxs0, idxs1), (idxd0, idxd1), (rows0, rows1),
            (isem0, isem1), (gsem0, gsem1), (ssem0, ssem1))
    rows_per = NPAD // NS
    pltpu.sync_copy(z_hbm, acc.at[pl.ds(sid * rows_per, rows_per)])
    plsc.subcore_barrier()
    wid = cid * NS + sid
    base = wid * (EPAD // (NC * NS))
    nblk = EPAD // (NC * NS) // BLK
    _edge_loop(src_hbm, dst_hbm, h_hbm, acc, bufs, base, nblk)
    plsc.subcore_barrier()
    sl = pl.ds(sid * rows_per, rows_per)

    @pl.when(cid == 0)
    def _():
        pltpu.sync_copy(acc.at[sl], o0_hbm.at[sl])

    @pl.when(cid == 1)
    def _():
        pltpu.sync_copy(acc.at[sl], o1_hbm.at[sl])


def _prop_scratch():
    return ([pltpu.VMEM_SHARED((NPAD, 128), jnp.float32)]
            + [pltpu.VMEM((BLK,), jnp.int32)] * 4
            + [pltpu.VMEM((BLK, 128), jnp.float32)] * 2
            + [pltpu.SemaphoreType.DMA] * 6)


def _prop_col(h0, h1, src, dst, zrows):
    return pl.kernel(
        _prop_col_kernel,
        out_type=(jax.ShapeDtypeStruct((NPAD, 128), jnp.float32),) * 2,
        mesh=plsc.VectorSubcoreMesh(**_MESH),
        scratch_types=_prop_scratch(),
        compiler_params=_SC_PARAMS,
    )(h0, h1, src, dst, zrows)


def _prop_edge(h, src, dst, zrows):
    return pl.kernel(
        _prop_edge_kernel,
        out_type=(jax.ShapeDtypeStruct((NPAD, 128), jnp.float32),) * 2,
        mesh=plsc.VectorSubcoreMesh(**_MESH),
        scratch_types=_prop_scratch(),
        compiler_params=_SC_PARAMS,
    )(h, src, dst, zrows)


RB = 256


def _rdeg(degp):
    return lax.rsqrt(jnp.maximum(jnp.sum(degp, axis=0), 1.0))


def _tc1_body(x_ref, w_ref, degp_ref, o0_ref, o1_ref):
    rdeg = _rdeg(degp_ref[...])
    h = jnp.dot(x_ref[...], w_ref[...], preferred_element_type=jnp.float32)
    hp = h * rdeg[:, None]
    o0_ref[...] = hp[:, :128]
    o1_ref[...] = hp[:, 128:]


def _tc1(x, w, degp):
    return pl.pallas_call(
        _tc1_body,
        grid=(NPAD // RB,),
        in_specs=[pl.BlockSpec((RB, D), lambda i: (i, 0)),
                  pl.BlockSpec((D, H1), lambda i: (0, 0)),
                  pl.BlockSpec((NC * NS, RB), lambda i: (0, i))],
        out_specs=[pl.BlockSpec((RB, 128), lambda i: (i, 0)),
                   pl.BlockSpec((RB, 128), lambda i: (i, 0))],
        out_shape=[jax.ShapeDtypeStruct((NPAD, 128), jnp.float32)] * 2,
    )(x, w, degp)


def _tc2_body(p0_ref, p1_ref, w_ref, degp_ref, o_ref):
    rdeg = _rdeg(degp_ref[...])
    hidden = jnp.concatenate([p0_ref[...], p1_ref[...]], axis=1)
    hidden = jnp.maximum(hidden * rdeg[:, None], 0.0)
    o_ref[...] = jnp.dot(hidden, w_ref[...],
                         preferred_element_type=jnp.float32) * rdeg[:, None]


def _tc2(p0, p1, w, degp):
    return pl.pallas_call(
        _tc2_body,
        grid=(NPAD // RB,),
        in_specs=[pl.BlockSpec((RB, 128), lambda i: (i, 0)),
                  pl.BlockSpec((RB, 128), lambda i: (i, 0)),
                  pl.BlockSpec((H1, 2 * H2), lambda i: (0, 0)),
                  pl.BlockSpec((NC * NS, RB), lambda i: (0, i))],
        out_specs=pl.BlockSpec((RB, 2 * H2), lambda i: (i, 0)),
        out_shape=jax.ShapeDtypeStruct((NPAD, 2 * H2), jnp.float32),
    )(p0, p1, w, degp)


def _tc3_body(q0_ref, q1_ref, degp_ref, eps_ref, mean_ref, std_ref, z_ref):
    rdeg = _rdeg(degp_ref[...])
    p = (q0_ref[...] + q1_ref[...]) * rdeg[:, None]
    mean = p[:, :H2]
    std = jnp.exp(p[:, H2:])
    mean_ref[...] = mean
    std_ref[...] = std
    z_ref[...] = mean + eps_ref[...] * std


def _tc3(q0, q1, degp, eps):
    return pl.pallas_call(
        _tc3_body,
        grid=(NPAD // RB,),
        in_specs=[pl.BlockSpec((RB, 128), lambda i: (i, 0)),
                  pl.BlockSpec((RB, 128), lambda i: (i, 0)),
                  pl.BlockSpec((NC * NS, RB), lambda i: (0, i)),
                  pl.BlockSpec((RB, H2), lambda i: (i, 0))],
        out_specs=[pl.BlockSpec((RB, H2), lambda i: (i, 0))] * 3,
        out_shape=[jax.ShapeDtypeStruct((NPAD, H2), jnp.float32)] * 3,
    )(q0, q1, degp, eps)


def _pad_edges(edge_index):
    pad = N + (jnp.arange(EPAD - E, dtype=jnp.int32) % (NPAD - N))
    src = jnp.concatenate([edge_index[0].astype(jnp.int32), pad])
    dst = jnp.concatenate([edge_index[1].astype(jnp.int32), pad])
    return src, dst


def _encoder(x, src, dst, degp, eps, W0, Wcat, zrows):
    xpad = jnp.pad(x, ((0, NPAD - N), (0, 0)))
    h1c0, h1c1 = _tc1(xpad, W0, degp)
    p1c0, p1c1 = _prop_col(h1c0, h1c1, src, dst, zrows)
    h2 = _tc2(p1c0, p1c1, Wcat, degp)
    q0, q1 = _prop_edge(h2, src, dst, zrows)
    epspad = jnp.pad(eps, ((0, NPAD - N), (0, 0)))
    mean, std, z = _tc3(q0, q1, degp, epspad)
    return mean[:N], std[:N], z[:N]


def kernel(xl0, xp0, edge_index_l, edge_index_p, eps_l, eps_p,
           W0_l, Wm_l, Ws_l, W0_p, Wm_p, Ws_p):
    src_l, dst_l = _pad_edges(edge_index_l)
    src_p, dst_p = _pad_edges(edge_index_p)
    zrows = jnp.zeros((NPAD // NS, 128), jnp.float32)
    degp_l, degp_p = _deg(dst_l, dst_p)
    Wcat_l = jnp.concatenate([Wm_l, Ws_l], axis=1)
    Wcat_p = jnp.concatenate([Wm_p, Ws_p], axis=1)
    hl, stdl, xl = _encoder(xl0, src_l, dst_l, degp_l, eps_l, W0_l, Wcat_l,
                            zrows)
    hp, stdp, xp = _encoder(xp0, src_p, dst_p, degp_p, eps_p, W0_p, Wcat_p,
                            zrows)
    return (hl, stdl, xl, hp, stdp, xp)


# speedup vs baseline: 20.0476x; 1.4240x over previous
import functools

import jax
import jax.numpy as jnp
from jax import lax
from jax.experimental import pallas as pl
from jax.experimental.pallas import tpu as pltpu
from jax.experimental.pallas import tpu_sc as plsc

N = 10000
NPAD = 10240
E = 320000
EPAD = 322560
D = 128
H1 = 256
H2 = 64
NC, NS = 2, 16
BLK = 112

_MESH = dict(core_axis_name="c", subcore_axis_name="s", num_cores=NC,
             num_subcores=NS)
_SC_PARAMS = pltpu.CompilerParams(needs_layout_passes=False)


def _deg_kernel(dst_a_hbm, dst_b_hbm, out_a_hbm, out_b_hbm, hist, idx):
    cid = lax.axis_index("c")
    sid = lax.axis_index("s")
    wid = cid * NS + sid
    ones = jnp.full((16,), 1.0, jnp.float32)
    zeros = jnp.zeros((16,), jnp.float32)

    def one_graph(dst_hbm, out_hbm):
        def zero(i, _):
            hist[pl.ds(i * 16, 16)] = zeros
            return 0
        lax.fori_loop(0, NPAD // 16, zero, 0)
        base = wid * (EPAD // (NC * NS))

        def block(b, _):
            pltpu.sync_copy(dst_hbm.at[pl.ds(base + b * 480, 480)], idx)
            for j in range(30):
                iv = idx[pl.ds(j * 16, 16)]
                plsc.addupdate_scatter(hist, [iv], ones)
            return 0
        lax.fori_loop(0, EPAD // (NC * NS) // 480, block, 0)
        pltpu.sync_copy(hist, out_hbm.at[wid])

    one_graph(dst_a_hbm, out_a_hbm)
    one_graph(dst_b_hbm, out_b_hbm)


def _deg(dst_a, dst_b):
    return pl.kernel(
        _deg_kernel,
        out_type=(jax.ShapeDtypeStruct((NC * NS, NPAD), jnp.float32),) * 2,
        mesh=plsc.VectorSubcoreMesh(**_MESH),
        scratch_types=[pltpu.VMEM((NPAD,), jnp.float32),
                       pltpu.VMEM((480,), jnp.int32)],
        compiler_params=_SC_PARAMS,
    )(dst_a, dst_b)


def _edge_loop(src_hbm, dst_hbm, h_hbm, acc, bufs, base, nblk):
    (idxs, idxd, rows, isem, gsem, ssem) = bufs

    def prefetch(b, s):
        off = base + b * BLK
        pltpu.async_copy(src_hbm.at[pl.ds(off, BLK)], idxs[s], isem[s])
        pltpu.async_copy(dst_hbm.at[pl.ds(off, BLK)], idxd[s], isem[s])

    def wait_prefetch(s):
        pltpu.make_async_copy(src_hbm.at[pl.ds(base, BLK)], idxs[s],
                              isem[s]).wait()
        pltpu.make_async_copy(dst_hbm.at[pl.ds(base, BLK)], idxd[s],
                              isem[s]).wait()

    def fire_gather(s):
        pltpu.async_copy(h_hbm.at[idxs[s]], rows[s], gsem[s])

    def wait_gather(s):
        pltpu.make_async_copy(h_hbm.at[idxs[s]], rows[s], gsem[s]).wait()

    def fire_scatter(s):
        pltpu.async_copy(rows[s], acc.at[idxd[s]], ssem[s], add=True)

    def wait_scatter(s):
        pltpu.make_async_copy(rows[s], acc.at[idxd[s]], ssem[s]).wait()

    def step(b, s, prev=True, prev2=True, pref=True):
        wait_prefetch(s)
        fire_gather(s)
        if prev:
            wait_gather((s + 2) % 3)
            fire_scatter((s + 2) % 3)
        if prev2:
            wait_scatter((s + 1) % 3)
        if pref:
            prefetch(b + 1, (s + 1) % 3)

    prefetch(0, 0)
    step(0, 0, prev=False, prev2=False)
    step(1, 1, prev2=False)
    ntrip = (nblk - 2) // 3

    def trip(m, _):
        b = 2 + 3 * m
        step(b, 2)
        step(b + 1, 0)
        step(b + 2, 1)
        return 0
    lax.fori_loop(0, ntrip, trip, 0)
    for b in range(2 + 3 * ntrip, nblk):
        step(b, b % 3, pref=(b + 1 < nblk))
    sl = (nblk - 1) % 3
    wait_gather(sl)
    fire_scatter(sl)
    wait_scatter((sl + 2) % 3)
    wait_scatter(sl)


def _prop_col_kernel(h0_hbm, h1_hbm, src_hbm, dst_hbm, z_hbm,
                     o0_hbm, o1_hbm, acc, idxs0, idxs1, idxs2, idxd0, idxd1,
                     idxd2, rows0, rows1, rows2, isem0, isem1, isem2, gsem0,
                     gsem1, gsem2, ssem0, ssem1, ssem2):
    cid = lax.axis_index("c")
    sid = lax.axis_index("s")
    bufs = ((idxs0, idxs1, idxs2), (idxd0, idxd1, idxd2),
            (rows0, rows1, rows2), (isem0, isem1, isem2),
            (gsem0, gsem1, gsem2), (ssem0, ssem1, ssem2))
    rows_per = NPAD // NS
    pltpu.sync_copy(z_hbm, acc.at[pl.ds(sid * rows_per, rows_per)])
    plsc.subcore_barrier()
    base = sid * (EPAD // NS)
    nblk = EPAD // NS // BLK

    @pl.when(cid == 0)
    def _():
        _edge_loop(src_hbm, dst_hbm, h0_hbm, acc, bufs, base, nblk)

    @pl.when(cid == 1)
    def _():
        _edge_loop(src_hbm, dst_hbm, h1_hbm, acc, bufs, base, nblk)

    plsc.subcore_barrier()
    sl = pl.ds(sid * rows_per, rows_per)

    @pl.when(cid == 0)
    def _():
        pltpu.sync_copy(acc.at[sl], o0_hbm.at[sl])

    @pl.when(cid == 1)
    def _():
        pltpu.sync_copy(acc.at[sl], o1_hbm.at[sl])


def _prop_edge_kernel(h_hbm, src_hbm, dst_hbm, z_hbm,
                      o0_hbm, o1_hbm, acc, idxs0, idxs1, idxs2, idxd0, idxd1,
                      idxd2, rows0, rows1, rows2, isem0, isem1, isem2, gsem0,
                      gsem1, gsem2, ssem0, ssem1, ssem2):
    cid = lax.axis_index("c")
    sid = lax.axis_index("s")
    bufs = ((idxs0, idxs1, idxs2), (idxd0, idxd1, idxd2),
            (rows0, rows1, rows2), (isem0, isem1, isem2),
            (gsem0, gsem1, gsem2), (ssem0, ssem1, ssem2))
    rows_per = NPAD // NS
    pltpu.sync_copy(z_hbm, acc.at[pl.ds(sid * rows_per, rows_per)])
    plsc.subcore_barrier()
    wid = cid * NS + sid
    base = wid * (EPAD // (NC * NS))
    nblk = EPAD // (NC * NS) // BLK
    _edge_loop(src_hbm, dst_hbm, h_hbm, acc, bufs, base, nblk)
    plsc.subcore_barrier()
    sl = pl.ds(sid * rows_per, rows_per)

    @pl.when(cid == 0)
    def _():
        pltpu.sync_copy(acc.at[sl], o0_hbm.at[sl])

    @pl.when(cid == 1)
    def _():
        pltpu.sync_copy(acc.at[sl], o1_hbm.at[sl])


def _prop_scratch():
    return ([pltpu.VMEM_SHARED((NPAD, 128), jnp.float32)]
            + [pltpu.VMEM((BLK,), jnp.int32)] * 6
            + [pltpu.VMEM((BLK, 128), jnp.float32)] * 3
            + [pltpu.SemaphoreType.DMA] * 9)


def _prop_col(h0, h1, src, dst, zrows):
    return pl.kernel(
        _prop_col_kernel,
        out_type=(jax.ShapeDtypeStruct((NPAD, 128), jnp.float32),) * 2,
        mesh=plsc.VectorSubcoreMesh(**_MESH),
        scratch_types=_prop_scratch(),
        compiler_params=_SC_PARAMS,
    )(h0, h1, src, dst, zrows)


def _prop_edge(h, src, dst, zrows):
    return pl.kernel(
        _prop_edge_kernel,
        out_type=(jax.ShapeDtypeStruct((NPAD, 128), jnp.float32),) * 2,
        mesh=plsc.VectorSubcoreMesh(**_MESH),
        scratch_types=_prop_scratch(),
        compiler_params=_SC_PARAMS,
    )(h, src, dst, zrows)


RB = 256


def _rdeg(degp):
    return lax.rsqrt(jnp.maximum(jnp.sum(degp, axis=0), 1.0))


def _tc1_body(x_ref, w_ref, degp_ref, o0_ref, o1_ref):
    rdeg = _rdeg(degp_ref[...])
    h = jnp.dot(x_ref[...], w_ref[...], preferred_element_type=jnp.float32)
    hp = h * rdeg[:, None]
    o0_ref[...] = hp[:, :128]
    o1_ref[...] = hp[:, 128:]


def _tc1(x, w, degp):
    return pl.pallas_call(
        _tc1_body,
        grid=(NPAD // RB,),
        in_specs=[pl.BlockSpec((RB, D), lambda i: (i, 0)),
                  pl.BlockSpec((D, H1), lambda i: (0, 0)),
                  pl.BlockSpec((NC * NS, RB), lambda i: (0, i))],
        out_specs=[pl.BlockSpec((RB, 128), lambda i: (i, 0)),
                   pl.BlockSpec((RB, 128), lambda i: (i, 0))],
        out_shape=[jax.ShapeDtypeStruct((NPAD, 128), jnp.float32)] * 2,
    )(x, w, degp)


def _tc2_body(p0_ref, p1_ref, w_ref, degp_ref, o_ref):
    rdeg = _rdeg(degp_ref[...])
    hidden = jnp.concatenate([p0_ref[...], p1_ref[...]], axis=1)
    hidden = jnp.maximum(hidden * rdeg[:, None], 0.0)
    o_ref[...] = jnp.dot(hidden, w_ref[...],
                         preferred_element_type=jnp.float32) * rdeg[:, None]


def _tc2(p0, p1, w, degp):
    return pl.pallas_call(
        _tc2_body,
        grid=(NPAD // RB,),
        in_specs=[pl.BlockSpec((RB, 128), lambda i: (i, 0)),
                  pl.BlockSpec((RB, 128), lambda i: (i, 0)),
                  pl.BlockSpec((H1, 2 * H2), lambda i: (0, 0)),
                  pl.BlockSpec((NC * NS, RB), lambda i: (0, i))],
        out_specs=pl.BlockSpec((RB, 2 * H2), lambda i: (i, 0)),
        out_shape=jax.ShapeDtypeStruct((NPAD, 2 * H2), jnp.float32),
    )(p0, p1, w, degp)


def _tc3_body(q0_ref, q1_ref, degp_ref, eps_ref, mean_ref, std_ref, z_ref):
    rdeg = _rdeg(degp_ref[...])
    p = (q0_ref[...] + q1_ref[...]) * rdeg[:, None]
    mean = p[:, :H2]
    std = jnp.exp(p[:, H2:])
    mean_ref[...] = mean
    std_ref[...] = std
    z_ref[...] = mean + eps_ref[...] * std


def _tc3(q0, q1, degp, eps):
    return pl.pallas_call(
        _tc3_body,
        grid=(NPAD // RB,),
        in_specs=[pl.BlockSpec((RB, 128), lambda i: (i, 0)),
                  pl.BlockSpec((RB, 128), lambda i: (i, 0)),
                  pl.BlockSpec((NC * NS, RB), lambda i: (0, i)),
                  pl.BlockSpec((RB, H2), lambda i: (i, 0))],
        out_specs=[pl.BlockSpec((RB, H2), lambda i: (i, 0))] * 3,
        out_shape=[jax.ShapeDtypeStruct((NPAD, H2), jnp.float32)] * 3,
    )(q0, q1, degp, eps)


def _pad_edges(edge_index):
    pad = N + (jnp.arange(EPAD - E, dtype=jnp.int32) % (NPAD - N))
    src = jnp.concatenate([edge_index[0].astype(jnp.int32), pad])
    dst = jnp.concatenate([edge_index[1].astype(jnp.int32), pad])
    return src, dst


def _encoder(x, src, dst, degp, eps, W0, Wcat, zrows):
    xpad = jnp.pad(x, ((0, NPAD - N), (0, 0)))
    h1c0, h1c1 = _tc1(xpad, W0, degp)
    p1c0, p1c1 = _prop_col(h1c0, h1c1, src, dst, zrows)
    h2 = _tc2(p1c0, p1c1, Wcat, degp)
    q0, q1 = _prop_edge(h2, src, dst, zrows)
    epspad = jnp.pad(eps, ((0, NPAD - N), (0, 0)))
    mean, std, z = _tc3(q0, q1, degp, epspad)
    return mean[:N], std[:N], z[:N]


def kernel(xl0, xp0, edge_index_l, edge_index_p, eps_l, eps_p,
           W0_l, Wm_l, Ws_l, W0_p, Wm_p, Ws_p):
    src_l, dst_l = _pad_edges(edge_index_l)
    src_p, dst_p = _pad_edges(edge_index_p)
    zrows = jnp.zeros((NPAD // NS, 128), jnp.float32)
    degp_l, degp_p = _deg(dst_l, dst_p)
    Wcat_l = jnp.concatenate([Wm_l, Ws_l], axis=1)
    Wcat_p = jnp.concatenate([Wm_p, Ws_p], axis=1)
    hl, stdl, xl = _encoder(xl0, src_l, dst_l, degp_l, eps_l, W0_l, Wcat_l,
                            zrows)
    hp, stdp, xp = _encoder(xp0, src_p, dst_p, degp_p, eps_p, W0_p, Wcat_p,
                            zrows)
    return (hl, stdl, xl, hp, stdp, xp)


# speedup vs baseline: 23.3354x; 1.1640x over previous
import jax
import jax.numpy as jnp
from jax import lax
from jax.experimental import pallas as pl
from jax.experimental.pallas import tpu as pltpu
from jax.experimental.pallas import tpu_sc as plsc

N = 10000
NPAD = 10240
E = 320000
EPAD = 322560
D = 128
H1 = 256
H2 = 64
NC, NS = 2, 16
BLK = 112

_MESH = dict(core_axis_name="c", subcore_axis_name="s", num_cores=NC,
             num_subcores=NS)
_SC_PARAMS = pltpu.CompilerParams(needs_layout_passes=False)


def _deg_kernel(dst_a_hbm, dst_b_hbm, out_hbm, hist, idx):
    cid = lax.axis_index("c")
    sid = lax.axis_index("s")
    wid = cid * NS + sid
    ones = jnp.full((16,), 1.0, jnp.float32)
    zeros = jnp.zeros((16,), jnp.float32)

    def one_graph(dst_hbm, out_row):
        def zero(i, _):
            hist[pl.ds(i * 16, 16)] = zeros
            return 0
        lax.fori_loop(0, NPAD // 16, zero, 0)
        base = wid * (EPAD // (NC * NS))

        def block(b, _):
            pltpu.sync_copy(dst_hbm.at[pl.ds(base + b * 480, 480)], idx)
            for j in range(30):
                iv = idx[pl.ds(j * 16, 16)]
                plsc.addupdate_scatter(hist, [iv], ones)
            return 0
        lax.fori_loop(0, EPAD // (NC * NS) // 480, block, 0)
        pltpu.sync_copy(hist, out_hbm.at[out_row])

    one_graph(dst_a_hbm, wid)
    one_graph(dst_b_hbm, 32 + wid)


def _deg(dst_a, dst_b):
    return pl.kernel(
        _deg_kernel,
        out_type=jax.ShapeDtypeStruct((2 * NC * NS, NPAD), jnp.float32),
        mesh=plsc.VectorSubcoreMesh(**_MESH),
        scratch_types=[pltpu.VMEM((NPAD,), jnp.float32),
                       pltpu.VMEM((480,), jnp.int32)],
        compiler_params=_SC_PARAMS,
    )(dst_a, dst_b)


def _edge_loop(src_hbm, dst_hbm, h_hbm, acc, bufs, base, nblk):
    (idxs, idxd, rows, isem, gsem, ssem) = bufs

    def prefetch(b, s):
        off = base + b * BLK
        pltpu.async_copy(src_hbm.at[pl.ds(off, BLK)], idxs[s], isem[s])
        pltpu.async_copy(dst_hbm.at[pl.ds(off, BLK)], idxd[s], isem[s])

    def wait_prefetch(s):
        pltpu.make_async_copy(src_hbm.at[pl.ds(base, BLK)], idxs[s],
                              isem[s]).wait()
        pltpu.make_async_copy(dst_hbm.at[pl.ds(base, BLK)], idxd[s],
                              isem[s]).wait()

    def fire_gather(s):
        pltpu.async_copy(h_hbm.at[idxs[s]], rows[s], gsem[s])

    def wait_gather(s):
        pltpu.make_async_copy(h_hbm.at[idxs[s]], rows[s], gsem[s]).wait()

    def fire_scatter(s):
        pltpu.async_copy(rows[s], acc.at[idxd[s]], ssem[s], add=True)

    def wait_scatter(s):
        pltpu.make_async_copy(rows[s], acc.at[idxd[s]], ssem[s]).wait()

    def step(b, s, prev=True, prev2=True, pref=True):
        wait_prefetch(s)
        fire_gather(s)
        if prev:
            wait_gather((s + 2) % 3)
            fire_scatter((s + 2) % 3)
        if prev2:
            wait_scatter((s + 1) % 3)
        if pref:
            prefetch(b + 1, (s + 1) % 3)

    prefetch(0, 0)
    step(0, 0, prev=False, prev2=False)
    step(1, 1, prev2=False)
    ntrip = (nblk - 2) // 3

    def trip(m, _):
        b = 2 + 3 * m
        step(b, 2)
        step(b + 1, 0)
        step(b + 2, 1)
        return 0
    lax.fori_loop(0, ntrip, trip, 0)
    for b in range(2 + 3 * ntrip, nblk):
        step(b, b % 3, pref=(b + 1 < nblk))
    sl = (nblk - 1) % 3
    wait_gather(sl)
    fire_scatter(sl)
    wait_scatter((sl + 2) % 3)
    wait_scatter(sl)


def _prop_kernel(h_hbm, src_l_hbm, dst_l_hbm, src_p_hbm, dst_p_hbm, z_hbm,
                 o_hbm, acc, idxs0, idxs1, idxs2, idxd0, idxd1, idxd2,
                 rows0, rows1, rows2, isem0, isem1, isem2, gsem0, gsem1,
                 gsem2, ssem0, ssem1, ssem2):
    cid = lax.axis_index("c")
    sid = lax.axis_index("s")
    bufs = ((idxs0, idxs1, idxs2), (idxd0, idxd1, idxd2),
            (rows0, rows1, rows2), (isem0, isem1, isem2),
            (gsem0, gsem1, gsem2), (ssem0, ssem1, ssem2))
    rows_per = NPAD // NS
    sl = pl.ds(sid * rows_per, rows_per)
    pltpu.sync_copy(z_hbm, acc.at[sl])
    plsc.subcore_barrier()
    base = sid * (EPAD // NS)
    nblk = EPAD // NS // BLK

    @pl.when(cid == 0)
    def _():
        _edge_loop(src_l_hbm, dst_l_hbm, h_hbm, acc, bufs, base, nblk)

    @pl.when(cid == 1)
    def _():
        _edge_loop(src_p_hbm, dst_p_hbm, h_hbm, acc, bufs, base, nblk)

    plsc.subcore_barrier()

    @pl.when(cid == 0)
    def _():
        pltpu.sync_copy(acc.at[sl], o_hbm.at[sl])

    @pl.when(cid == 1)
    def _():
        pltpu.sync_copy(acc.at[sl],
                        o_hbm.at[pl.ds(NPAD + sid * rows_per, rows_per)])


def _prop(h, src_l, dst_l, src_p, dst_p, zrows):
    return pl.kernel(
        _prop_kernel,
        out_type=jax.ShapeDtypeStruct((2 * NPAD, 128), jnp.float32),
        mesh=plsc.VectorSubcoreMesh(**_MESH),
        scratch_types=([pltpu.VMEM_SHARED((NPAD, 128), jnp.float32)]
                       + [pltpu.VMEM((BLK,), jnp.int32)] * 6
                       + [pltpu.VMEM((BLK, 128), jnp.float32)] * 3
                       + [pltpu.SemaphoreType.DMA] * 9),
        compiler_params=_SC_PARAMS,
    )(h, src_l, dst_l, src_p, dst_p, zrows)


RB = 256
GB = NPAD // RB


def _rdeg(degp):
    return lax.rsqrt(jnp.maximum(jnp.sum(degp, axis=0), 1.0))


def _tc_scale_body(x_ref, degp_ref, o_ref):
    o_ref[...] = x_ref[...] * _rdeg(degp_ref[...])[:, None]


def _tc_scale(x, degp):
    return pl.pallas_call(
        _tc_scale_body,
        grid=(2, GB),
        in_specs=[pl.BlockSpec((RB, D), lambda g, i: (g * GB + i, 0)),
                  pl.BlockSpec((NC * NS, RB), lambda g, i: (g, i))],
        out_specs=pl.BlockSpec((RB, D), lambda g, i: (g * GB + i, 0)),
        out_shape=jax.ShapeDtypeStruct((2 * NPAD, D), jnp.float32),
    )(x, degp)


def _tc_mid_body(p_ref, w0_ref, wc_ref, degp_ref, o_ref):
    rdeg = _rdeg(degp_ref[...])
    h = jnp.dot(p_ref[...], w0_ref[...], preferred_element_type=jnp.float32)
    hidden = jnp.maximum(h * rdeg[:, None], 0.0)
    o_ref[...] = jnp.dot(hidden, wc_ref[...],
                         preferred_element_type=jnp.float32) * rdeg[:, None]


def _tc_mid(p, w0, wc, degp):
    return pl.pallas_call(
        _tc_mid_body,
        grid=(2, GB),
        in_specs=[pl.BlockSpec((RB, D), lambda g, i: (g * GB + i, 0)),
                  pl.BlockSpec((D, H1), lambda g, i: (g, 0)),
                  pl.BlockSpec((H1, 2 * H2), lambda g, i: (g, 0)),
                  pl.BlockSpec((NC * NS, RB), lambda g, i: (g, i))],
        out_specs=pl.BlockSpec((RB, 2 * H2), lambda g, i: (g * GB + i, 0)),
        out_shape=jax.ShapeDtypeStruct((2 * NPAD, 2 * H2), jnp.float32),
    )(p, w0, wc, degp)


def _tc_out_body(q_ref, degp_ref, eps_ref, mean_ref, std_ref, z_ref):
    p = q_ref[...] * _rdeg(degp_ref[...])[:, None]
    mean = p[:, :H2]
    std = jnp.exp(p[:, H2:])
    mean_ref[...] = mean
    std_ref[...] = std
    z_ref[...] = mean + eps_ref[...] * std


def _tc_out(q, degp, eps):
    return pl.pallas_call(
        _tc_out_body,
        grid=(2, GB),
        in_specs=[pl.BlockSpec((RB, 2 * H2), lambda g, i: (g * GB + i, 0)),
                  pl.BlockSpec((NC * NS, RB), lambda g, i: (g, i)),
                  pl.BlockSpec((RB, H2), lambda g, i: (g * GB + i, 0))],
        out_specs=[pl.BlockSpec((RB, H2), lambda g, i: (g * GB + i, 0))] * 3,
        out_shape=[jax.ShapeDtypeStruct((2 * NPAD, H2), jnp.float32)] * 3,
    )(q, degp, eps)


def _pad_edges(edge_index, src_off):
    pad = N + (jnp.arange(EPAD - E, dtype=jnp.int32) % (NPAD - N))
    src = jnp.concatenate([edge_index[0].astype(jnp.int32), pad]) + src_off
    dst = jnp.concatenate([edge_index[1].astype(jnp.int32), pad])
    return src, dst


def _pad_cat(a, b):
    return jnp.concatenate([jnp.pad(a, ((0, NPAD - N), (0, 0))),
                            jnp.pad(b, ((0, NPAD - N), (0, 0)))])


def kernel(xl0, xp0, edge_index_l, edge_index_p, eps_l, eps_p,
           W0_l, Wm_l, Ws_l, W0_p, Wm_p, Ws_p):
    src_l, dst_l = _pad_edges(edge_index_l, 0)
    src_p, dst_p = _pad_edges(edge_index_p, NPAD)
    zrows = jnp.zeros((NPAD // NS, 128), jnp.float32)
    degp = _deg(dst_l, dst_p)
    x = _pad_cat(xl0, xp0)
    xs = _tc_scale(x, degp)
    p1 = _prop(xs, src_l, dst_l, src_p, dst_p, zrows)
    w0 = jnp.concatenate([W0_l, W0_p], axis=0)
    wc = jnp.concatenate([jnp.concatenate([Wm_l, Ws_l], axis=1),
                          jnp.concatenate([Wm_p, Ws_p], axis=1)], axis=0)
    h2 = _tc_mid(p1, w0, wc, degp)
    q = _prop(h2, src_l, dst_l, src_p, dst_p, zrows)
    eps = _pad_cat(eps_l, eps_p)
    mean, std, z = _tc_out(q, degp, eps)
    return (mean[:N], std[:N], z[:N],
            mean[NPAD:NPAD + N], std[NPAD:NPAD + N], z[NPAD:NPAD + N])
